# Initial kernel scaffold; baseline (speedup 1.0000x reference)
import jax
import jax.numpy as jnp
from jax.experimental import pallas as pl


def kernel(X, A, params):
    raise NotImplementedError("write your pallas kernel here")


# speedup vs baseline: 1.0830x; 1.0830x over previous
import functools

import jax
import jax.numpy as jnp
from jax.experimental import pallas as pl

_N = 10000
_AVG_DEG = 32


def _proj_body(x_ref, wl_ref, wr_ref, hl_ref, hr_ref):
    x = x_ref[...]
    hl_ref[...] = jnp.dot(x, wl_ref[...], preferred_element_type=jnp.float32)
    hr_ref[...] = jnp.dot(x, wr_ref[...], preferred_element_type=jnp.float32)


@functools.partial(jax.jit, static_argnums=())
def _proj(x, wl, wr):
    n, din = x.shape
    dout = wl.shape[1]
    blk = 2000
    grid = (n // blk,)
    return pl.pallas_call(
        _proj_body,
        grid=grid,
        in_specs=[
            pl.BlockSpec((blk, din), lambda i: (i, 0)),
            pl.BlockSpec((din, dout), lambda i: (0, 0)),
            pl.BlockSpec((din, dout), lambda i: (0, 0)),
        ],
        out_specs=[
            pl.BlockSpec((blk, dout), lambda i: (i, 0)),
            pl.BlockSpec((blk, dout), lambda i: (i, 0)),
        ],
        out_shape=[
            jax.ShapeDtypeStruct((n, dout), jnp.float32),
            jax.ShapeDtypeStruct((n, dout), jnp.float32),
        ],
    )(x, wl, wr)


def _gat_layer(x, src, dst, p, n):
    hl, hr = _proj(x, p["Wl"], p["Wr"])
    e = jax.nn.leaky_relu(hl[dst] + hr[src], negative_slope=0.2) @ p["att"]
    m = jax.ops.segment_max(e, dst, num_segments=n)
    m = jnp.where(jnp.isfinite(m), m, 0.0)
    ex = jnp.exp(e - m[dst])
    den = jax.ops.segment_sum(ex, dst, num_segments=n)
    alpha = ex / (den[dst] + 1e-16)
    return jax.ops.segment_sum(alpha[:, None] * hr[src], dst, num_segments=n)


def kernel(X, A, params):
    n = X.shape[0]
    nz = jnp.nonzero(A, size=_N * _AVG_DEG, fill_value=_N)
    src, dst = nz[0], nz[1]
    H = X
    for layer in params["enc"]:
        H = _gat_layer(H, src, dst, layer, n)
    Z = H
    for layer in params["dec"]:
        Z = _gat_layer(Z, src, dst, layer, n)
    return Z


# speedup vs baseline: 2.1255x; 1.9626x over previous
import dataclasses
import functools

import jax
import jax.numpy as jnp
from jax import lax
from jax.experimental import pallas as pl
from jax.experimental.pallas import tpu as pltpu
from jax.experimental.pallas import tpu_sc as plsc

_N = 10000
_AVG_DEG = 32
_E = _N * _AVG_DEG
_W = 32
_SEG = 313
_PAD = 320
_PN = _W * _PAD
_CAP = _E
_K = 128
_L = 16


def _proj_body(x_ref, wl_ref, wr_ref, hl_ref, hr_ref):
    x = x_ref[...]
    hl_ref[...] = jnp.dot(x, wl_ref[...], preferred_element_type=jnp.float32)
    hr_ref[...] = jnp.dot(x, wr_ref[...], preferred_element_type=jnp.float32)


def _proj(x, wl, wr):
    n, din = x.shape
    dout = wl.shape[1]
    blk = 1024
    return pl.pallas_call(
        _proj_body,
        grid=(n // blk,),
        in_specs=[
            pl.BlockSpec((blk, din), lambda i: (i, 0)),
            pl.BlockSpec((din, dout), lambda i: (0, 0)),
            pl.BlockSpec((din, dout), lambda i: (0, 0)),
        ],
        out_specs=[
            pl.BlockSpec((blk, dout), lambda i: (i, 0)),
            pl.BlockSpec((blk, dout), lambda i: (i, 0)),
        ],
        out_shape=[
            jax.ShapeDtypeStruct((n, dout), jnp.float32),
            jax.ShapeDtypeStruct((n, dout), jnp.float32),
        ],
    )(x, wl, wr)


@functools.partial(jax.jit, static_argnames=("dout",))
def _sc_layer(hl, hr, att, deg, edges, dout):
    r_blk = dout // _L
    mesh = plsc.VectorSubcoreMesh(core_axis_name="c", subcore_axis_name="s")
    cp = pltpu.CompilerParams()
    if "needs_layout_passes" in pltpu.CompilerParams.__dataclass_fields__:
        cp = dataclasses.replace(cp, needs_layout_passes=False)

    @functools.partial(
        pl.kernel,
        compiler_params=cp,
        out_type=jax.ShapeDtypeStruct((_PN, dout), jnp.float32),
        mesh=mesh,
        scratch_types=[
            pltpu.VMEM((_K, dout), jnp.float32),
            pltpu.VMEM((_K,), jnp.int32),
            pltpu.VMEM((16, dout), jnp.float32),
            pltpu.VMEM((16, dout), jnp.float32),
            pltpu.VMEM((dout,), jnp.float32),
            pltpu.VMEM((_PAD + _L,), jnp.int32),
        ],
    )
    def layer(hl_hbm, hr_hbm, att_hbm, deg_hbm, edges_hbm, out_hbm,
              hr_buf, idx_buf, hl_buf, out_buf, att_buf, deg_buf):
        w = lax.axis_index("c") * 16 + lax.axis_index("s")
        node0 = w * _PAD
        ebase = w * _CAP

        pltpu.sync_copy(att_hbm, att_buf)
        pltpu.sync_copy(deg_hbm.at[pl.ds(node0, _PAD)],
                        deg_buf.at[pl.ds(0, _PAD)])
        att_v = [att_buf[pl.ds(_L * j, _L)] for j in range(r_blk)]

        def seg_body(d, seg_base):
            @pl.when(lax.rem(d, 16) == 0)
            def _():
                pltpu.sync_copy(
                    hl_hbm.at[pl.ds(pl.multiple_of(node0 + d, 16), 16)],
                    hl_buf)

            drow = lax.rem(d, 16)
            hl_v = [hl_buf[drow, pl.ds(_L * j, _L)] for j in range(r_blk)]
            r = deg_buf[pl.ds(d, _L)][0]

            def edge_body(k, carry):
                m, den, acc = carry
                g = seg_base + k

                @pl.when(lax.rem(g, _K) == 0)
                def _():
                    pltpu.sync_copy(
                        edges_hbm.at[pl.ds(ebase + (g // _K) * _K, _K)],
                        idx_buf)
                    pltpu.sync_copy(hr_hbm.at[idx_buf], hr_buf)

                row = lax.rem(g, _K)
                hr_v = [hr_buf[row, pl.ds(_L * j, _L)] for j in range(r_blk)]
                s = jnp.zeros((_L,), jnp.float32)
                for j in range(r_blk):
                    t = hl_v[j] + hr_v[j]
                    t = jnp.maximum(t, 0.0) + 0.2 * jnp.minimum(t, 0.0)
                    s = s + t * att_v[j]
                e = jnp.sum(s)
                new_m = jnp.maximum(m, e)
                scale = jnp.exp(jnp.full((_L,), m - new_m, jnp.float32))
                wv = jnp.exp(jnp.full((_L,), e - new_m, jnp.float32))
                den = den * scale + wv
                acc = [acc[j] * scale + wv * hr_v[j] for j in range(r_blk)]
                return new_m, den, acc

            init = (jnp.float32(-jnp.inf),
                    jnp.zeros((_L,), jnp.float32),
                    [jnp.zeros((_L,), jnp.float32) for _ in range(r_blk)])
            m, den, acc = lax.fori_loop(0, r, edge_body, init)

            ok = den > 0.0
            for j in range(r_blk):
                out_buf[drow, pl.ds(_L * j, _L)] = jnp.where(
                    ok, acc[j] / jnp.where(ok, den, 1.0), 0.0)

            @pl.when(lax.rem(d, 16) == 15)
            def _():
                pltpu.sync_copy(
                    out_buf,
                    out_hbm.at[pl.ds(pl.multiple_of(node0 + d - 15, 16), 16)])

            return seg_base + r

        lax.fori_loop(0, _PAD, seg_body, jnp.int32(0))

    return layer(hl, hr, att, deg, edges)


def _edge_prep(A):
    nz = jnp.nonzero(A, size=_E, fill_value=_N)
    src, dst = nz[0].astype(jnp.int32), nz[1].astype(jnp.int32)
    srcp = (src // _SEG) * _PAD + src % _SEG
    key = dst * 16384 + srcp
    ks = jnp.sort(key)
    dsts = ks >> 14
    srcs = ks & 16383
    off = jnp.searchsorted(dsts, jnp.arange(_W * _SEG + 1, dtype=jnp.int32),
                           side="left").astype(jnp.int32)
    deg = off[1:] - off[:-1]
    deg_pad = jnp.pad(deg.reshape(_W, _SEG), ((0, 0), (0, _PAD - _SEG))
                      ).reshape(_PN)
    we = dsts // _SEG
    starts = off[jnp.arange(_W) * _SEG]
    pos = we * _CAP + (jnp.arange(_E, dtype=jnp.int32) - starts[we])
    edge_arr = jnp.zeros((_W * _CAP,), jnp.int32).at[pos].set(srcs)
    return deg_pad, edge_arr


def _pad_nodes(X):
    X = jnp.pad(X, ((0, _W * _SEG - _N), (0, 0)))
    X = jnp.pad(X.reshape(_W, _SEG, -1), ((0, 0), (0, _PAD - _SEG), (0, 0)))
    return X.reshape(_PN, -1)


def _unpad_nodes(Z):
    Z = Z.reshape(_W, _PAD, -1)[:, :_SEG].reshape(_W * _SEG, -1)
    return Z[:_N]


def kernel(X, A, params):
    deg_pad, edge_arr = _edge_prep(A)
    H = _pad_nodes(X)
    for layer in params["enc"] + params["dec"]:
        din, dout = layer["Wl"].shape
        dip = max(din, 128)
        dop = max(dout, 128)
        wl = jnp.pad(layer["Wl"], ((0, dip - din), (0, dop - dout)))
        wr = jnp.pad(layer["Wr"], ((0, dip - din), (0, dop - dout)))
        att = jnp.pad(layer["att"], (0, dop - dout))
        hl, hr = _proj(H, wl, wr)
        H = _sc_layer(hl, hr, att, deg_pad, edge_arr, dout=dop)
    return _unpad_nodes(H)


# speedup vs baseline: 9.2650x; 4.3589x over previous
import dataclasses
import functools

import jax
import jax.numpy as jnp
from jax import lax
from jax.experimental import pallas as pl
from jax.experimental.pallas import tpu as pltpu
from jax.experimental.pallas import tpu_sc as plsc

_N = 10000
_W = 32
_PAD = 320
_PN = _W * _PAD
_CAP = _N * 32
_K = 128
_L = 16
_NW = 640


def _sc_params():
    cp = pltpu.CompilerParams()
    if "needs_layout_passes" in pltpu.CompilerParams.__dataclass_fields__:
        cp = dataclasses.replace(cp, needs_layout_passes=False)
    return cp


def _proj_body(x_ref, wl_ref, wr_ref, hl_ref, hr_ref):
    x = x_ref[...]
    hl_ref[...] = jnp.dot(x, wl_ref[...], preferred_element_type=jnp.float32)
    hr_ref[...] = jnp.dot(x, wr_ref[...], preferred_element_type=jnp.float32)


def _proj(x, wl, wr):
    n, din = x.shape
    dout = wl.shape[1]
    blk = 1024
    return pl.pallas_call(
        _proj_body,
        grid=(n // blk,),
        in_specs=[
            pl.BlockSpec((blk, din), lambda i: (i, 0)),
            pl.BlockSpec((din, dout), lambda i: (0, 0)),
            pl.BlockSpec((din, dout), lambda i: (0, 0)),
        ],
        out_specs=[
            pl.BlockSpec((blk, dout), lambda i: (i, 0)),
            pl.BlockSpec((blk, dout), lambda i: (i, 0)),
        ],
        out_shape=[
            jax.ShapeDtypeStruct((n, dout), jnp.float32),
            jax.ShapeDtypeStruct((n, dout), jnp.float32),
        ],
    )(x, wl, wr)


def _pack_body(a_ref, pw_ref, out_ref):
    a = a_ref[...].astype(jnp.bfloat16)
    w16 = lax.dot_general(a, pw_ref[...],
                          (((0,), (1,)), ((), ())),
                          preferred_element_type=jnp.float32)
    out_ref[...] = w16.astype(jnp.int32)


def _bitpack(A):
    dblk = 128
    pow_const = jnp.where(
        (jnp.arange(_N)[None, :] // 16) == jnp.arange(_NW)[:, None],
        jnp.exp2(jnp.arange(_N, dtype=jnp.float32) % 16)[None, :],
        0.0).astype(jnp.bfloat16)
    return pl.pallas_call(
        _pack_body,
        grid=(pl.cdiv(_N, dblk),),
        in_specs=[
            pl.BlockSpec((_N, dblk), lambda i: (0, i)),
            pl.BlockSpec((_NW, _N), lambda i: (0, 0)),
        ],
        out_specs=pl.BlockSpec((dblk, _NW), lambda i: (i, 0)),
        out_shape=jax.ShapeDtypeStruct((_N, _NW), jnp.int32),
    )(A, pow_const)


@jax.jit
def _sc_extract(bm):
    mesh = plsc.VectorSubcoreMesh(core_axis_name="c", subcore_axis_name="s")

    @functools.partial(
        pl.kernel,
        out_type=[
            jax.ShapeDtypeStruct((_W * _CAP,), jnp.int32),
            jax.ShapeDtypeStruct((_PN,), jnp.int32),
        ],
        mesh=mesh,
        compiler_params=_sc_params(),
        scratch_types=[
            pltpu.VMEM((16, _NW), jnp.int32),
            pltpu.VMEM((256,), jnp.int32),
            pltpu.VMEM((_PAD,), jnp.int32),
        ],
    )
    def extract(bm_hbm, edges_hbm, deg_hbm, bm_buf, ring, deg_buf):
        w = lax.axis_index("c") * 16 + lax.axis_index("s")
        node0 = w * _PAD
        ebase = w * _CAP
        lanes = lax.iota(jnp.int32, _L)

        for t in range(16):
            ring[pl.ds(16 * t, 16)] = jnp.zeros((16,), jnp.int32)

        def row_body(d, carry):
            cur, nfl, deg_vec = carry
            valid = node0 + d < _N

            @pl.when(valid & (lax.rem(d, 16) == 0))
            def _():
                pltpu.sync_copy(
                    bm_hbm.at[pl.ds(pl.multiple_of(node0 + d, 16), 16)],
                    bm_buf)

            drow = lax.rem(d, 16)
            n0 = 128 * nfl + cur

            def vreg_body(v, carry2):
                cur, nfl = carry2
                wvec = bm_buf[drow, pl.ds(_L * v, _L)]

                def bit_body(st):
                    wv, cur, nfl = st
                    low = wv & (-wv)
                    e = (lax.shift_right_logical(
                        plsc.bitcast(low.astype(jnp.float32), jnp.int32),
                        23) - 127)
                    widx = _L * v + lanes
                    src = widx * 16 + e
                    mask = wv != 0
                    plsc.store_compressed(ring.at[pl.ds(cur, _L)], src,
                                          mask=mask)
                    cnt = plsc.all_reduce_population_count(mask)[0]
                    cur = cur + cnt
                    wv = wv ^ low

                    @pl.when(cur >= 128)
                    def _():
                        pltpu.sync_copy(
                            ring.at[pl.ds(0, 128)],
                            edges_hbm.at[pl.ds(ebase + nfl * 128, 128)])
                        for t in range(8):
                            ring[pl.ds(16 * t, 16)] = \
                                ring[pl.ds(128 + 16 * t, 16)]

                    nfl = jnp.where(cur >= 128, nfl + 1, nfl)
                    cur = jnp.where(cur >= 128, cur - 128, cur)
                    return wv, cur, nfl

                wv, cur, nfl = lax.while_loop(
                    lambda st: jnp.any(st[0] != 0), bit_body,
                    (wvec, cur, nfl))
                return cur, nfl

            cur, nfl = lax.cond(
                valid,
                lambda c: lax.fori_loop(0, _NW // _L, vreg_body, c),
                lambda c: c, (cur, nfl))

            rowcnt = (128 * nfl + cur) - n0
            deg_vec = deg_vec + jnp.where(lanes == drow, rowcnt, 0)

            @pl.when(lax.rem(d, 16) == 15)
            def _():
                deg_buf[pl.ds(pl.multiple_of(d - 15, 16), 16)] = deg_vec

            deg_vec = jnp.where(lax.rem(d, 16) == 15,
                                jnp.zeros((_L,), jnp.int32), deg_vec)
            return cur, nfl, deg_vec

        cur, nfl, _unused = lax.fori_loop(
            0, _PAD, row_body,
            (jnp.int32(0), jnp.int32(0), jnp.zeros((_L,), jnp.int32)))

        @pl.when(cur > 0)
        def _():
            pltpu.sync_copy(ring.at[pl.ds(0, 128)],
                            edges_hbm.at[pl.ds(ebase + nfl * 128, 128)])

        pltpu.sync_copy(deg_buf, deg_hbm.at[pl.ds(node0, _PAD)])

    return extract(bm)


@functools.partial(jax.jit, static_argnames=("dout",))
def _sc_layer(hl, hr, att, deg, edges, dout):
    r_blk = dout // _L
    mesh = plsc.VectorSubcoreMesh(core_axis_name="c", subcore_axis_name="s")

    @functools.partial(
        pl.kernel,
        out_type=jax.ShapeDtypeStruct((_PN, dout), jnp.float32),
        mesh=mesh,
        compiler_params=_sc_params(),
        scratch_types=[
            pltpu.VMEM((_K, dout), jnp.float32),
            pltpu.VMEM((_K,), jnp.int32),
            pltpu.VMEM((16, dout), jnp.float32),
            pltpu.VMEM((16, dout), jnp.float32),
            pltpu.VMEM((dout,), jnp.float32),
            pltpu.VMEM((_PAD + _L,), jnp.int32),
        ],
    )
    def layer(hl_hbm, hr_hbm, att_hbm, deg_hbm, edges_hbm, out_hbm,
              hr_buf, idx_buf, hl_buf, out_buf, att_buf, deg_buf):
        w = lax.axis_index("c") * 16 + lax.axis_index("s")
        node0 = w * _PAD
        ebase = w * _CAP

        pltpu.sync_copy(att_hbm, att_buf)
        pltpu.sync_copy(deg_hbm.at[pl.ds(node0, _PAD)],
                        deg_buf.at[pl.ds(0, _PAD)])
        att_v = [att_buf[pl.ds(_L * j, _L)] for j in range(r_blk)]

        def seg_body(d, seg_base):
            @pl.when(lax.rem(d, 16) == 0)
            def _():
                pltpu.sync_copy(
                    hl_hbm.at[pl.ds(pl.multiple_of(node0 + d, 16), 16)],
                    hl_buf)

            drow = lax.rem(d, 16)
            hl_v = [hl_buf[drow, pl.ds(_L * j, _L)] for j in range(r_blk)]
            r = deg_buf[pl.ds(d, _L)][0]

            def edge_body(k, carry):
                m, den, acc = carry
                g = seg_base + k

                @pl.when(lax.rem(g, _K) == 0)
                def _():
                    pltpu.sync_copy(
                        edges_hbm.at[pl.ds(ebase + (g // _K) * _K, _K)],
                        idx_buf)
                    pltpu.sync_copy(hr_hbm.at[idx_buf], hr_buf)

                row = lax.rem(g, _K)
                hr_v = [hr_buf[row, pl.ds(_L * j, _L)] for j in range(r_blk)]
                s = jnp.zeros((_L,), jnp.float32)
                for j in range(r_blk):
                    t = hl_v[j] + hr_v[j]
                    t = jnp.maximum(t, 0.0) + 0.2 * jnp.minimum(t, 0.0)
                    s = s + t * att_v[j]
                e = jnp.sum(s)
                new_m = jnp.maximum(m, e)
                scale = jnp.exp(jnp.full((_L,), m - new_m, jnp.float32))
                wv = jnp.exp(jnp.full((_L,), e - new_m, jnp.float32))
                den = den * scale + wv
                acc = [acc[j] * scale + wv * hr_v[j] for j in range(r_blk)]
                return new_m, den, acc

            init = (jnp.float32(-jnp.inf),
                    jnp.zeros((_L,), jnp.float32),
                    [jnp.zeros((_L,), jnp.float32) for _ in range(r_blk)])
            m, den, acc = lax.fori_loop(0, r, edge_body, init)

            ok = den > 0.0
            for j in range(r_blk):
                out_buf[drow, pl.ds(_L * j, _L)] = jnp.where(
                    ok, acc[j] / jnp.where(ok, den, 1.0), 0.0)

            @pl.when(lax.rem(d, 16) == 15)
            def _():
                pltpu.sync_copy(
                    out_buf,
                    out_hbm.at[pl.ds(pl.multiple_of(node0 + d - 15, 16), 16)])

            return seg_base + r

        lax.fori_loop(0, _PAD, seg_body, jnp.int32(0))

    return layer(hl, hr, att, deg, edges)


def kernel(X, A, params):
    bm = _bitpack(A)
    edge_arr, deg_pad = _sc_extract(bm)
    H = jnp.pad(X, ((0, _PN - _N), (0, 0)))
    for layer in params["enc"] + params["dec"]:
        din, dout = layer["Wl"].shape
        dip = max(din, 128)
        dop = max(dout, 128)
        wl = jnp.pad(layer["Wl"], ((0, dip - din), (0, dop - dout)))
        wr = jnp.pad(layer["Wr"], ((0, dip - din), (0, dop - dout)))
        att = jnp.pad(layer["att"], (0, dop - dout))
        hl, hr = _proj(H, wl, wr)
        H = _sc_layer(hl, hr, att, deg_pad, edge_arr, dout=dop)
    return H[:_N]


# speedup vs baseline: 10.6956x; 1.1544x over previous
import dataclasses
import functools

import jax
import jax.numpy as jnp
from jax import lax
from jax.experimental import pallas as pl
from jax.experimental.pallas import tpu as pltpu
from jax.experimental.pallas import tpu_sc as plsc

_N = 10000
_W = 32
_PAD = 320
_PN = _W * _PAD
_CAP = _N * 32
_K = 128
_L = 16
_NW = 640


def _sc_params():
    cp = pltpu.CompilerParams()
    if "needs_layout_passes" in pltpu.CompilerParams.__dataclass_fields__:
        cp = dataclasses.replace(cp, needs_layout_passes=False)
    return cp


def _proj_body(x_ref, wl_ref, wr_ref, hl_ref, hr_ref):
    x = x_ref[...]
    hl_ref[...] = jnp.dot(x, wl_ref[...], preferred_element_type=jnp.float32)
    hr_ref[...] = jnp.dot(x, wr_ref[...], preferred_element_type=jnp.float32)


def _proj(x, wl, wr):
    n, din = x.shape
    dout = wl.shape[1]
    blk = 1024
    return pl.pallas_call(
        _proj_body,
        grid=(n // blk,),
        in_specs=[
            pl.BlockSpec((blk, din), lambda i: (i, 0)),
            pl.BlockSpec((din, dout), lambda i: (0, 0)),
            pl.BlockSpec((din, dout), lambda i: (0, 0)),
        ],
        out_specs=[
            pl.BlockSpec((blk, dout), lambda i: (i, 0)),
            pl.BlockSpec((blk, dout), lambda i: (i, 0)),
        ],
        out_shape=[
            jax.ShapeDtypeStruct((n, dout), jnp.float32),
            jax.ShapeDtypeStruct((n, dout), jnp.float32),
        ],
    )(x, wl, wr)


def _pack_body(a_ref, pw_ref, out_ref):
    a = a_ref[...].astype(jnp.bfloat16)
    w16 = lax.dot_general(a, pw_ref[...],
                          (((0,), (1,)), ((), ())),
                          preferred_element_type=jnp.float32)
    out_ref[...] = w16.astype(jnp.int32)


def _bitpack(A):
    dblk = 128
    pow_const = jnp.where(
        (jnp.arange(_N)[None, :] // 16) == jnp.arange(_NW)[:, None],
        jnp.exp2(jnp.arange(_N, dtype=jnp.float32) % 16)[None, :],
        0.0).astype(jnp.bfloat16)
    return pl.pallas_call(
        _pack_body,
        grid=(pl.cdiv(_N, dblk),),
        in_specs=[
            pl.BlockSpec((_N, dblk), lambda i: (0, i)),
            pl.BlockSpec((_NW, _N), lambda i: (0, 0)),
        ],
        out_specs=pl.BlockSpec((dblk, _NW), lambda i: (i, 0)),
        out_shape=jax.ShapeDtypeStruct((_N, _NW), jnp.int32),
    )(A, pow_const)


@jax.jit
def _sc_extract(bm):
    mesh = plsc.VectorSubcoreMesh(core_axis_name="c", subcore_axis_name="s")

    @functools.partial(
        pl.kernel,
        out_type=[
            jax.ShapeDtypeStruct((_W * _CAP,), jnp.int32),
            jax.ShapeDtypeStruct((_PN,), jnp.int32),
        ],
        mesh=mesh,
        compiler_params=_sc_params(),
        scratch_types=[
            pltpu.VMEM((16, _NW), jnp.int32),
            pltpu.VMEM((256,), jnp.int32),
            pltpu.VMEM((_PAD,), jnp.int32),
        ],
    )
    def extract(bm_hbm, edges_hbm, deg_hbm, bm_buf, ring, deg_buf):
        w = lax.axis_index("c") * 16 + lax.axis_index("s")
        node0 = w * _PAD
        ebase = w * _CAP
        lanes = lax.iota(jnp.int32, _L)

        for t in range(16):
            ring[pl.ds(16 * t, 16)] = jnp.zeros((16,), jnp.int32)

        def row_body(d, carry):
            cur, nfl, deg_vec = carry
            valid = node0 + d < _N

            @pl.when(valid & (lax.rem(d, 16) == 0))
            def _():
                pltpu.sync_copy(
                    bm_hbm.at[pl.ds(pl.multiple_of(node0 + d, 16), 16)],
                    bm_buf)

            drow = lax.rem(d, 16)
            n0 = 128 * nfl + cur

            def vreg_body(v, carry2):
                cur, nfl = carry2
                wvec = bm_buf[drow, pl.ds(_L * v, _L)]

                def bit_body(st):
                    wv, cur, nfl = st
                    low = wv & (-wv)
                    e = (lax.shift_right_logical(
                        plsc.bitcast(low.astype(jnp.float32), jnp.int32),
                        23) - 127)
                    widx = _L * v + lanes
                    src = widx * 16 + e
                    mask = wv != 0
                    plsc.store_compressed(ring.at[pl.ds(cur, _L)], src,
                                          mask=mask)
                    cnt = plsc.all_reduce_population_count(mask)[0]
                    cur = cur + cnt
                    wv = wv ^ low

                    @pl.when(cur >= 128)
                    def _():
                        pltpu.sync_copy(
                            ring.at[pl.ds(0, 128)],
                            edges_hbm.at[pl.ds(ebase + nfl * 128, 128)])
                        for t in range(8):
                            ring[pl.ds(16 * t, 16)] = \
                                ring[pl.ds(128 + 16 * t, 16)]

                    nfl = jnp.where(cur >= 128, nfl + 1, nfl)
                    cur = jnp.where(cur >= 128, cur - 128, cur)
                    return wv, cur, nfl

                wv, cur, nfl = lax.while_loop(
                    lambda st: jnp.any(st[0] != 0), bit_body,
                    (wvec, cur, nfl))
                return cur, nfl

            cur, nfl = lax.cond(
                valid,
                lambda c: lax.fori_loop(0, _NW // _L, vreg_body, c),
                lambda c: c, (cur, nfl))

            rowcnt = (128 * nfl + cur) - n0
            deg_vec = deg_vec + jnp.where(lanes == drow, rowcnt, 0)

            @pl.when(lax.rem(d, 16) == 15)
            def _():
                deg_buf[pl.ds(pl.multiple_of(d - 15, 16), 16)] = deg_vec

            deg_vec = jnp.where(lax.rem(d, 16) == 15,
                                jnp.zeros((_L,), jnp.int32), deg_vec)
            return cur, nfl, deg_vec

        cur, nfl, _unused = lax.fori_loop(
            0, _PAD, row_body,
            (jnp.int32(0), jnp.int32(0), jnp.zeros((_L,), jnp.int32)))

        @pl.when(cur > 0)
        def _():
            pltpu.sync_copy(ring.at[pl.ds(0, 128)],
                            edges_hbm.at[pl.ds(ebase + nfl * 128, 128)])

        pltpu.sync_copy(deg_buf, deg_hbm.at[pl.ds(node0, _PAD)])

    return extract(bm)


@functools.partial(jax.jit, static_argnames=("dout",))
def _sc_layer(hl, hr, att, deg, edges, dout):
    r_blk = dout // _L
    mesh = plsc.VectorSubcoreMesh(core_axis_name="c", subcore_axis_name="s")

    @functools.partial(
        pl.kernel,
        out_type=jax.ShapeDtypeStruct((_PN, dout), jnp.float32),
        mesh=mesh,
        compiler_params=_sc_params(),
        scratch_types=[
            pltpu.VMEM((2 * _K, dout), jnp.float32),
            pltpu.VMEM((2 * _K,), jnp.int32),
            pltpu.VMEM((16, dout), jnp.float32),
            pltpu.VMEM((16, dout), jnp.float32),
            pltpu.VMEM((dout,), jnp.float32),
            pltpu.VMEM((_PAD + _L,), jnp.int32),
            pltpu.SemaphoreType.DMA,
            pltpu.SemaphoreType.DMA,
        ],
    )
    def layer(hl_hbm, hr_hbm, att_hbm, deg_hbm, edges_hbm, out_hbm,
              hr_buf, idx_buf, hl_buf, out_buf, att_buf, deg_buf,
              sem0, sem1):
        w = lax.axis_index("c") * 16 + lax.axis_index("s")
        node0 = w * _PAD
        ebase = w * _CAP

        pltpu.sync_copy(att_hbm, att_buf)
        pltpu.sync_copy(deg_hbm.at[pl.ds(node0, _PAD)],
                        deg_buf.at[pl.ds(0, _PAD)])
        att_v = [att_buf[pl.ds(_L * j, _L)] for j in range(r_blk)]

        dacc = jnp.zeros((_L,), jnp.int32)
        for t in range(_PAD // _L):
            dacc = dacc + deg_buf[pl.ds(_L * t, _L)]
        cnt = jnp.sum(dacc)

        def _gather(par, sem):
            pltpu.async_copy(
                hr_hbm.at[idx_buf.at[pl.ds(par * _K, _K)]],
                hr_buf.at[pl.ds(par * _K, _K)], sem)

        def _gwait(par, sem):
            pltpu.make_async_copy(
                hr_hbm.at[idx_buf.at[pl.ds(par * _K, _K)]],
                hr_buf.at[pl.ds(par * _K, _K)], sem).wait()

        @pl.when(cnt > 0)
        def _():
            pltpu.sync_copy(edges_hbm.at[pl.ds(ebase, _K)],
                            idx_buf.at[pl.ds(0, _K)])
            _gather(0, sem0)

        def seg_body(d, seg_base):
            @pl.when(lax.rem(d, 16) == 0)
            def _():
                pltpu.sync_copy(
                    hl_hbm.at[pl.ds(pl.multiple_of(node0 + d, 16), 16)],
                    hl_buf)

            drow = lax.rem(d, 16)
            hl_v = [hl_buf[drow, pl.ds(_L * j, _L)] for j in range(r_blk)]
            r = deg_buf[pl.ds(d, _L)][0]

            def edge_body(k, carry):
                m, den, acc = carry
                g = seg_base + k

                @pl.when(lax.rem(g, _K) == 0)
                def _():
                    c = g // _K
                    par = lax.rem(c, 2)
                    npar = 1 - par

                    @pl.when(par == 0)
                    def _():
                        _gwait(0, sem0)

                    @pl.when(par == 1)
                    def _():
                        _gwait(1, sem1)

                    @pl.when((c + 1) * _K < cnt)
                    def _():
                        pltpu.sync_copy(
                            edges_hbm.at[pl.ds(ebase + (c + 1) * _K, _K)],
                            idx_buf.at[pl.ds(npar * _K, _K)])

                        @pl.when(par == 0)
                        def _():
                            _gather(1, sem1)

                        @pl.when(par == 1)
                        def _():
                            _gather(0, sem0)

                row = lax.rem(g, 2 * _K)
                hr_v = [hr_buf[row, pl.ds(_L * j, _L)] for j in range(r_blk)]
                s = jnp.zeros((_L,), jnp.float32)
                for j in range(r_blk):
                    t = hl_v[j] + hr_v[j]
                    t = jnp.maximum(t, 0.0) + 0.2 * jnp.minimum(t, 0.0)
                    s = s + t * att_v[j]
                e = jnp.sum(s)
                new_m = jnp.maximum(m, e)
                scale = jnp.exp(jnp.full((_L,), m - new_m, jnp.float32))
                wv = jnp.exp(jnp.full((_L,), e - new_m, jnp.float32))
                den = den * scale + wv
                acc = [acc[j] * scale + wv * hr_v[j] for j in range(r_blk)]
                return new_m, den, acc

            init = (jnp.float32(-jnp.inf),
                    jnp.zeros((_L,), jnp.float32),
                    [jnp.zeros((_L,), jnp.float32) for _ in range(r_blk)])
            m, den, acc = lax.fori_loop(0, r, edge_body, init)

            ok = den > 0.0
            for j in range(r_blk):
                out_buf[drow, pl.ds(_L * j, _L)] = jnp.where(
                    ok, acc[j] / jnp.where(ok, den, 1.0), 0.0)

            @pl.when(lax.rem(d, 16) == 15)
            def _():
                pltpu.sync_copy(
                    out_buf,
                    out_hbm.at[pl.ds(pl.multiple_of(node0 + d - 15, 16), 16)])

            return seg_base + r

        lax.fori_loop(0, _PAD, seg_body, jnp.int32(0))

    return layer(hl, hr, att, deg, edges)


def kernel(X, A, params):
    bm = _bitpack(A)
    edge_arr, deg_pad = _sc_extract(bm)
    H = jnp.pad(X, ((0, _PN - _N), (0, 0)))
    for layer in params["enc"] + params["dec"]:
        din, dout = layer["Wl"].shape
        dip = max(din, 128)
        dop = max(dout, 128)
        wl = jnp.pad(layer["Wl"], ((0, dip - din), (0, dop - dout)))
        wr = jnp.pad(layer["Wr"], ((0, dip - din), (0, dop - dout)))
        att = jnp.pad(layer["att"], (0, dop - dout))
        hl, hr = _proj(H, wl, wr)
        H = _sc_layer(hl, hr, att, deg_pad, edge_arr, dout=dop)
    return H[:_N]
